# baseline (speedup 1.0000x reference)
import functools

import jax
import jax.numpy as jnp
from jax import lax
from jax.experimental import pallas as pl
from jax.experimental.pallas import tpu as pltpu
from jax.experimental.pallas import tpu_sc as plsc

B = 16384
N = 200
D = 128
Y_SIZE = 20
V = 400
DW = D // 2
VS = 65
NW = 32
BPW = B // NW
L = 16
NG = 13
XB = 8
_N0 = [min(L * j, N - L) for j in range(NG)]


def _tec_body(xy_hbm, emb_hbm, out_hbm,
              emb_v, xy_v0, xy_v1, ob0, ob1, sx0, sx1, so0, so1):
    wid = lax.axis_index("s") * 2 + lax.axis_index("c")
    base = wid * BPW

    pltpu.sync_copy(emb_hbm, emb_v)

    xy_bufs = (xy_v0, xy_v1)
    out_bufs = (ob0, ob1)
    xy_sems = (sx0, sx1)
    out_sems = (so0, so1)
    lane = lax.iota(jnp.int32, L)

    xw = 2 * N * XB
    pltpu.async_copy(xy_hbm.at[pl.ds(base * 2 * N, xw)], xy_v0, sx0)
    pltpu.async_copy(xy_hbm.at[pl.ds(base * 2 * N + xw, xw)], xy_v1, sx1)

    def gbody(g, carry):
        for q in range(2):
            blk = g * 2 + q
            xyv = xy_bufs[q]
            pltpu.make_async_copy(
                xy_hbm.at[pl.ds(base * 2 * N, xw)], xyv, xy_sems[q]).wait()
            for m in range(XB):
                bl = blk * XB + m
                b = base + bl
                k = m % 2
                obuf = out_bufs[k]

                pos = []
                for j in range(NG):
                    xi = lane * 2 + (2 * _N0[j] + 2 * N * m)
                    xv = plsc.load_gather(xyv, [xi])
                    yv = plsc.load_gather(xyv, [xi + 1])
                    pos.append(xv * (Y_SIZE * VS) + yv * VS)

                @pl.when(bl >= 2)
                def _():
                    pltpu.make_async_copy(
                        obuf, out_hbm.at[b], out_sems[k]).wait()

                @plsc.parallel_loop(0, DW, unroll=4)
                def _(d2):
                    dv = lax.broadcast(d2, (L,))
                    for j in range(NG):
                        w = plsc.load_gather(emb_v, [pos[j] + dv])
                        wb = plsc.bitcast(w, jnp.bfloat16)
                        lo, hi = plsc.unpack(
                            wb, format=plsc.PackFormat.INTERLEAVED)
                        obuf[2 * d2, pl.ds(_N0[j], L)] = lo
                        obuf[2 * d2 + 1, pl.ds(_N0[j], L)] = hi

                pltpu.async_copy(obuf, out_hbm.at[b], out_sems[k])

            @pl.when(blk + 2 < BPW // XB)
            def _():
                pltpu.async_copy(
                    xy_hbm.at[pl.ds((base + (blk + 2) * XB) * 2 * N, xw)],
                    xyv, xy_sems[q])
        return carry

    lax.fori_loop(0, BPW // (2 * XB), gbody, 0)

    pltpu.make_async_copy(ob0, out_hbm.at[base + BPW - 2], so0).wait()
    pltpu.make_async_copy(ob1, out_hbm.at[base + BPW - 1], so1).wait()


@jax.jit
def _impl(xyf, embf):
    run = functools.partial(
        pl.kernel,
        out_type=jax.ShapeDtypeStruct((B, D, N), jnp.float32),
        mesh=plsc.VectorSubcoreMesh(core_axis_name="c", subcore_axis_name="s"),
        compiler_params=pltpu.CompilerParams(needs_layout_passes=False),
        scratch_types=[
            pltpu.VMEM((V * VS,), jnp.int32),
            pltpu.VMEM((XB * 2 * N,), jnp.int32),
            pltpu.VMEM((XB * 2 * N,), jnp.int32),
            pltpu.VMEM((D, N), jnp.float32),
            pltpu.VMEM((D, N), jnp.float32),
            pltpu.SemaphoreType.DMA,
            pltpu.SemaphoreType.DMA,
            pltpu.SemaphoreType.DMA,
            pltpu.SemaphoreType.DMA,
        ],
    )(_tec_body)
    return run(xyf, embf)


def kernel(xy, embedding):
    xyf = xy.reshape(-1)
    packed = lax.bitcast_convert_type(
        embedding.astype(jnp.bfloat16).reshape(V, DW, 2), jnp.int32)
    embf = jnp.pad(packed, ((0, 0), (0, VS - DW))).reshape(-1)
    return _impl(xyf, embf)


# speedup vs baseline: 2.7215x; 2.7215x over previous
import functools

import jax
import jax.numpy as jnp
from jax import lax
from jax.experimental import pallas as pl
from jax.experimental.pallas import tpu as pltpu
from jax.experimental.pallas import tpu_sc as plsc

B = 16384
N = 200
D = 128
Y_SIZE = 20
V = 400
DW = D // 2
VS = 65
NW = 32
BPW = B // NW
L = 16
NG = 13
_N0 = [min(L * j, N - L) for j in range(NG)]


def _tec_body(xy_hbm, emb_hbm, out_hbm,
              emb_v, xy_v0, xy_v1, ob0, ob1, sx0, sx1, so0, so1):
    wid = lax.axis_index("s") * 2 + lax.axis_index("c")
    base = wid * BPW

    pltpu.sync_copy(emb_hbm, emb_v)

    xy_bufs = (xy_v0, xy_v1)
    out_bufs = (ob0, ob1)
    xy_sems = (sx0, sx1)
    out_sems = (so0, so1)
    lane = lax.iota(jnp.int32, L)

    pltpu.async_copy(xy_hbm.at[base], xy_v0, sx0)
    pltpu.async_copy(xy_hbm.at[base + 1], xy_v1, sx1)

    def gbody(g, carry):
        for k in range(2):
            bl = g * 2 + k
            b = base + bl
            xyv = xy_bufs[k]
            obuf = out_bufs[k]

            pltpu.make_async_copy(xy_hbm.at[b], xyv, xy_sems[k]).wait()

            pos = []
            for j in range(NG):
                xi = lane * 2 + (2 * _N0[j])
                xv = plsc.load_gather(xyv, [xi])
                yv = plsc.load_gather(xyv, [xi + 1])
                pos.append(xv * (Y_SIZE * VS) + yv * VS)

            @pl.when(bl + 2 < BPW)
            def _():
                pltpu.async_copy(xy_hbm.at[b + 2], xyv, xy_sems[k])

            @pl.when(bl >= 2)
            def _():
                pltpu.make_async_copy(obuf, out_hbm.at[b], out_sems[k]).wait()

            @plsc.parallel_loop(0, DW, unroll=4)
            def _(d2):
                dv = lax.broadcast(d2, (L,))
                for j in range(NG):
                    w = plsc.load_gather(emb_v, [pos[j] + dv])
                    wb = plsc.bitcast(w, jnp.bfloat16)
                    lo, hi = plsc.unpack(wb, format=plsc.PackFormat.INTERLEAVED)
                    obuf[2 * d2, pl.ds(_N0[j], L)] = lo
                    obuf[2 * d2 + 1, pl.ds(_N0[j], L)] = hi

            pltpu.async_copy(obuf, out_hbm.at[b], out_sems[k])
        return carry

    lax.fori_loop(0, BPW // 2, gbody, 0)

    pltpu.make_async_copy(ob0, out_hbm.at[base + BPW - 2], so0).wait()
    pltpu.make_async_copy(ob1, out_hbm.at[base + BPW - 1], so1).wait()


@jax.jit
def _impl(xyf, embf):
    run = functools.partial(
        pl.kernel,
        out_type=jax.ShapeDtypeStruct((B, D, N), jnp.float32),
        mesh=plsc.VectorSubcoreMesh(core_axis_name="c", subcore_axis_name="s"),
        compiler_params=pltpu.CompilerParams(needs_layout_passes=False),
        scratch_types=[
            pltpu.VMEM((V * VS,), jnp.int32),
            pltpu.VMEM((2 * N,), jnp.int32),
            pltpu.VMEM((2 * N,), jnp.int32),
            pltpu.VMEM((D, N), jnp.float32),
            pltpu.VMEM((D, N), jnp.float32),
            pltpu.SemaphoreType.DMA,
            pltpu.SemaphoreType.DMA,
            pltpu.SemaphoreType.DMA,
            pltpu.SemaphoreType.DMA,
        ],
    )(_tec_body)
    return run(xyf, embf)


def kernel(xy, embedding):
    xyf = xy.reshape(B, 2 * N)
    packed = lax.bitcast_convert_type(
        embedding.astype(jnp.bfloat16).reshape(V, DW, 2), jnp.int32)
    embf = jnp.pad(packed, ((0, 0), (0, VS - DW))).reshape(-1)
    return _impl(xyf, embf)
